# baseline (speedup 1.0000x reference)
import functools

import jax
import jax.numpy as jnp
from jax import lax
from jax.experimental import pallas as pl
from jax.experimental.pallas import tpu as pltpu
from jax.experimental.pallas import tpu_sc as plsc


TILE = 32


def _route_body(x_ref, gw_ref, dest_ref, te_ref, tot_ref, T, E):
    S = x_ref.shape[0]
    NT_PAD = te_ref.shape[0]
    x = x_ref[...]
    logits = jnp.dot(x, gw_ref[...], preferred_element_type=jnp.float32)

    e_iota = lax.broadcasted_iota(jnp.int32, (S, E), 1)
    m = jnp.max(logits, axis=1, keepdims=True)
    eid = jnp.min(jnp.where(logits == m, e_iota, E), axis=1, keepdims=True)
    onehot = (e_iota == eid).astype(jnp.float32)

    C = 256
    NCH = S // C
    rc = lax.broadcasted_iota(jnp.int32, (C, C), 0)
    cc = lax.broadcasted_iota(jnp.int32, (C, C), 1)
    ltri_c = (cc <= rc).astype(jnp.float32)
    intra = [
        jnp.dot(ltri_c, onehot[i * C : (i + 1) * C], preferred_element_type=jnp.float32)
        for i in range(NCH)
    ]
    totals = jnp.concatenate([b[C - 1 : C] for b in intra], axis=0)
    hr = lax.broadcasted_iota(jnp.int32, (NCH, NCH), 0)
    hc = lax.broadcasted_iota(jnp.int32, (NCH, NCH), 1)
    strict_lt = (hc < hr).astype(jnp.float32)
    coffs = jnp.dot(
        strict_lt, totals, preferred_element_type=jnp.float32
    )
    incl = jnp.concatenate(
        [intra[i] + coffs[i : i + 1] for i in range(NCH)], axis=0
    )
    rank = jnp.sum(incl * onehot, axis=1, keepdims=True) - 1.0

    counts = jnp.sum(totals, axis=0, keepdims=True)
    ntiles = jnp.floor((counts + (T - 1)) * (1.0 / T))
    er = lax.broadcasted_iota(jnp.int32, (E, E), 0)
    ec = lax.broadcasted_iota(jnp.int32, (E, E), 1)
    utri = (er <= ec).astype(jnp.float32)
    cum_incl = jnp.dot(ntiles, utri, preferred_element_type=jnp.float32)
    cum_excl = cum_incl - ntiles
    tile_base = cum_excl * float(T)

    tb_token = jnp.sum(onehot * tile_base, axis=1, keepdims=True)
    dest_ref[...] = (tb_token + rank).astype(jnp.int32)

    g_col = lax.broadcasted_iota(jnp.int32, (NT_PAD, 1), 0).astype(jnp.float32)
    past = (g_col >= cum_incl).astype(jnp.float32)
    te = jnp.sum(past, axis=1, keepdims=True)
    te_ref[...] = jnp.minimum(te, float(E - 1)).astype(jnp.int32)
    tot_ref[...] = jnp.max(cum_incl, axis=1, keepdims=True).astype(jnp.int32)


def _ff_body(
    te_ref, tot_ref, x_ref, g_ref, b_ref, w1_ref, b1_ref, w2_ref, b2_ref, o_ref
):
    del te_ref

    @pl.when(pl.program_id(0) < tot_ref[0])
    def _():
        x = x_ref[...]
        mu = jnp.mean(x, axis=1, keepdims=True)
        xc = x - mu
        var = jnp.mean(xc * xc, axis=1, keepdims=True)
        ln = xc * lax.rsqrt(var + 1e-5) * g_ref[0] + b_ref[0]
        h = jnp.dot(ln, w1_ref[0], preferred_element_type=jnp.float32) + b1_ref[0]
        h = 0.5 * h * (1.0 + lax.erf(h * (2.0 ** -0.5)))
        o_ref[...] = (
            jnp.dot(h, w2_ref[0], preferred_element_type=jnp.float32) + b2_ref[0]
        )


def _make_sc_scatter(S, D, P, chunk, nw):
    mesh = plsc.VectorSubcoreMesh(core_axis_name="c", subcore_axis_name="s")

    @functools.partial(
        pl.kernel,
        mesh=mesh,
        out_type=jax.ShapeDtypeStruct((P, D), jnp.float32),
        scratch_types=[
            pltpu.VMEM((chunk,), jnp.int32),
            pltpu.VMEM((chunk, D), jnp.float32),
            pltpu.SemaphoreType.DMA,
            pltpu.SemaphoreType.DMA,
        ],
    )
    def scatter_k(x_hbm, dest_hbm, padded_hbm, idx_v, rows_v, sem_i, sem_r):
        wid = lax.axis_index("s") * 2 + lax.axis_index("c")
        base = wid * chunk
        cp_i = pltpu.async_copy(dest_hbm.at[pl.ds(base, chunk)], idx_v, sem_i)
        cp_r = pltpu.async_copy(x_hbm.at[pl.ds(base, chunk)], rows_v, sem_r)
        cp_i.wait()
        cp_r.wait()
        pltpu.async_copy(rows_v, padded_hbm.at[idx_v], sem_r).wait()

    return scatter_k


def _make_sc_gather(S, D, P, chunk, nw):
    mesh = plsc.VectorSubcoreMesh(core_axis_name="c", subcore_axis_name="s")

    @functools.partial(
        pl.kernel,
        mesh=mesh,
        out_type=jax.ShapeDtypeStruct((S, D), jnp.float32),
        scratch_types=[
            pltpu.VMEM((chunk,), jnp.int32),
            pltpu.VMEM((chunk, D), jnp.float32),
            pltpu.SemaphoreType.DMA,
        ],
    )
    def gather_k(padded_hbm, dest_hbm, out_hbm, idx_v, rows_v, sem):
        wid = lax.axis_index("s") * 2 + lax.axis_index("c")
        base = wid * chunk
        pltpu.sync_copy(dest_hbm.at[pl.ds(base, chunk)], idx_v)
        pltpu.async_copy(padded_hbm.at[idx_v], rows_v, sem).wait()
        pltpu.sync_copy(rows_v, out_hbm.at[pl.ds(base, chunk)])

    return gather_k


def kernel(x, gate_W, ln_g, ln_b, W1, b1, W2, b2):
    B, S_, D = x.shape
    S = B * S_
    E = gate_W.shape[1]
    H = W1.shape[2]
    T = TILE
    NT = S // T + E
    NT_PAD = ((NT + 7) // 8) * 8
    P = NT * T

    x_sq = x.reshape(S, D)

    dest2d, te2d, tot2d = pl.pallas_call(
        functools.partial(_route_body, T=T, E=E),
        out_shape=(
            jax.ShapeDtypeStruct((S, 1), jnp.int32),
            jax.ShapeDtypeStruct((NT_PAD, 1), jnp.int32),
            jax.ShapeDtypeStruct((1, 1), jnp.int32),
        ),
    )(x_sq, gate_W)
    dest = dest2d.reshape(S)
    tile_expert = te2d.reshape(NT_PAD)[:NT]
    total_tiles = tot2d.reshape(1)

    nw = 32
    chunk = S // nw
    padded_x = _make_sc_scatter(S, D, P, chunk, nw)(x_sq, dest)

    def _gmap(g, te, tot):
        return (jnp.minimum(g, tot[0] - 1), 0)

    def _emap3(g, te, tot):
        return (te[g], 0, 0)

    grid_spec = pltpu.PrefetchScalarGridSpec(
        num_scalar_prefetch=2,
        grid=(NT,),
        in_specs=[
            pl.BlockSpec((T, D), _gmap),
            pl.BlockSpec((1, 1, D), _emap3),
            pl.BlockSpec((1, 1, D), _emap3),
            pl.BlockSpec((1, D, H), _emap3),
            pl.BlockSpec((1, 1, H), _emap3),
            pl.BlockSpec((1, H, D), _emap3),
            pl.BlockSpec((1, 1, D), _emap3),
        ],
        out_specs=pl.BlockSpec((T, D), _gmap),
    )
    padded_out = pl.pallas_call(
        _ff_body,
        grid_spec=grid_spec,
        out_shape=jax.ShapeDtypeStruct((P, D), jnp.float32),
        compiler_params=pltpu.CompilerParams(
            dimension_semantics=("arbitrary",),
        ),
    )(
        tile_expert,
        total_tiles,
        padded_x,
        ln_g.reshape(E, 1, D),
        ln_b.reshape(E, 1, D),
        W1,
        b1.reshape(E, 1, H),
        W2,
        b2.reshape(E, 1, D),
    )

    out = _make_sc_gather(S, D, P, chunk, nw)(padded_out, dest)
    return out.reshape(x.shape)


# speedup vs baseline: 1.1932x; 1.1932x over previous
import functools

import jax
import jax.numpy as jnp
from jax import lax
from jax.experimental import pallas as pl
from jax.experimental.pallas import tpu as pltpu
from jax.experimental.pallas import tpu_sc as plsc


TILE = 64


def _route_body(x_ref, gw_ref, dest_ref, te_ref, tot_ref, T, E):
    S = x_ref.shape[0]
    NT_PAD = te_ref.shape[0]
    x = x_ref[...]
    logits = jnp.dot(x, gw_ref[...], preferred_element_type=jnp.float32)

    e_iota = lax.broadcasted_iota(jnp.int32, (S, E), 1)
    m = jnp.max(logits, axis=1, keepdims=True)
    eid = jnp.min(jnp.where(logits == m, e_iota, E), axis=1, keepdims=True)
    onehot = (e_iota == eid).astype(jnp.float32)

    C = 256
    NCH = S // C
    rc = lax.broadcasted_iota(jnp.int32, (C, C), 0)
    cc = lax.broadcasted_iota(jnp.int32, (C, C), 1)
    ltri_c = (cc <= rc).astype(jnp.float32)
    intra = [
        jnp.dot(ltri_c, onehot[i * C : (i + 1) * C], preferred_element_type=jnp.float32)
        for i in range(NCH)
    ]
    totals = jnp.concatenate([b[C - 1 : C] for b in intra], axis=0)
    hr = lax.broadcasted_iota(jnp.int32, (NCH, NCH), 0)
    hc = lax.broadcasted_iota(jnp.int32, (NCH, NCH), 1)
    strict_lt = (hc < hr).astype(jnp.float32)
    coffs = jnp.dot(
        strict_lt, totals, preferred_element_type=jnp.float32
    )
    incl = jnp.concatenate(
        [intra[i] + coffs[i : i + 1] for i in range(NCH)], axis=0
    )
    rank = jnp.sum(incl * onehot, axis=1, keepdims=True) - 1.0

    counts = jnp.sum(totals, axis=0, keepdims=True)
    ntiles = jnp.floor((counts + (T - 1)) * (1.0 / T))
    er = lax.broadcasted_iota(jnp.int32, (E, E), 0)
    ec = lax.broadcasted_iota(jnp.int32, (E, E), 1)
    utri = (er <= ec).astype(jnp.float32)
    cum_incl = jnp.dot(ntiles, utri, preferred_element_type=jnp.float32)
    cum_excl = cum_incl - ntiles
    tile_base = cum_excl * float(T)

    tb_token = jnp.sum(onehot * tile_base, axis=1, keepdims=True)
    dest_ref[...] = (tb_token + rank).astype(jnp.int32)

    g_col = lax.broadcasted_iota(jnp.int32, (NT_PAD, 1), 0).astype(jnp.float32)
    past = (g_col >= cum_incl).astype(jnp.float32)
    te = jnp.sum(past, axis=1, keepdims=True)
    te_ref[...] = jnp.minimum(te, float(E - 1)).astype(jnp.int32)
    tot_ref[...] = jnp.max(cum_incl, axis=1, keepdims=True).astype(jnp.int32)


def _ff_body(
    te_ref, tot_ref, x_ref, g_ref, b_ref, w1_ref, b1_ref, w2_ref, b2_ref, o_ref
):
    del te_ref

    @pl.when(pl.program_id(0) < tot_ref[0])
    def _():
        x = x_ref[...]
        mu = jnp.mean(x, axis=1, keepdims=True)
        xc = x - mu
        var = jnp.mean(xc * xc, axis=1, keepdims=True)
        ln = xc * lax.rsqrt(var + 1e-5) * g_ref[0] + b_ref[0]
        h = jnp.dot(ln, w1_ref[0], preferred_element_type=jnp.float32) + b1_ref[0]
        h = 0.5 * h * (1.0 + lax.erf(h * (2.0 ** -0.5)))
        o_ref[...] = (
            jnp.dot(h, w2_ref[0], preferred_element_type=jnp.float32) + b2_ref[0]
        )


def _make_sc_scatter(S, D, P, chunk, nw):
    mesh = plsc.VectorSubcoreMesh(core_axis_name="c", subcore_axis_name="s")

    @functools.partial(
        pl.kernel,
        mesh=mesh,
        out_type=jax.ShapeDtypeStruct((P, D), jnp.float32),
        scratch_types=[
            pltpu.VMEM((chunk,), jnp.int32),
            pltpu.VMEM((chunk, D), jnp.float32),
            pltpu.SemaphoreType.DMA,
            pltpu.SemaphoreType.DMA,
        ],
    )
    def scatter_k(x_hbm, dest_hbm, padded_hbm, idx_v, rows_v, sem_i, sem_r):
        wid = lax.axis_index("s") * 2 + lax.axis_index("c")
        base = wid * chunk
        cp_i = pltpu.async_copy(dest_hbm.at[pl.ds(base, chunk)], idx_v, sem_i)
        cp_r = pltpu.async_copy(x_hbm.at[pl.ds(base, chunk)], rows_v, sem_r)
        cp_i.wait()
        cp_r.wait()
        pltpu.async_copy(rows_v, padded_hbm.at[idx_v], sem_r).wait()

    return scatter_k


def _make_sc_gather(S, D, P, chunk, nw):
    mesh = plsc.VectorSubcoreMesh(core_axis_name="c", subcore_axis_name="s")

    @functools.partial(
        pl.kernel,
        mesh=mesh,
        out_type=jax.ShapeDtypeStruct((S, D), jnp.float32),
        scratch_types=[
            pltpu.VMEM((chunk,), jnp.int32),
            pltpu.VMEM((chunk, D), jnp.float32),
            pltpu.SemaphoreType.DMA,
        ],
    )
    def gather_k(padded_hbm, dest_hbm, out_hbm, idx_v, rows_v, sem):
        wid = lax.axis_index("s") * 2 + lax.axis_index("c")
        base = wid * chunk
        pltpu.sync_copy(dest_hbm.at[pl.ds(base, chunk)], idx_v)
        pltpu.async_copy(padded_hbm.at[idx_v], rows_v, sem).wait()
        pltpu.sync_copy(rows_v, out_hbm.at[pl.ds(base, chunk)])

    return gather_k


def kernel(x, gate_W, ln_g, ln_b, W1, b1, W2, b2):
    B, S_, D = x.shape
    S = B * S_
    E = gate_W.shape[1]
    H = W1.shape[2]
    T = TILE
    NT = S // T + E
    NT_PAD = ((NT + 7) // 8) * 8
    P = NT * T

    x_sq = x.reshape(S, D)

    dest2d, te2d, tot2d = pl.pallas_call(
        functools.partial(_route_body, T=T, E=E),
        out_shape=(
            jax.ShapeDtypeStruct((S, 1), jnp.int32),
            jax.ShapeDtypeStruct((NT_PAD, 1), jnp.int32),
            jax.ShapeDtypeStruct((1, 1), jnp.int32),
        ),
    )(x_sq, gate_W)
    dest = dest2d.reshape(S)
    tile_expert = te2d.reshape(NT_PAD)[:NT]
    total_tiles = tot2d.reshape(1)

    nw = 32
    chunk = S // nw
    padded_x = _make_sc_scatter(S, D, P, chunk, nw)(x_sq, dest)

    def _gmap(g, te, tot):
        return (jnp.minimum(g, tot[0] - 1), 0)

    def _emap3(g, te, tot):
        return (te[g], 0, 0)

    grid_spec = pltpu.PrefetchScalarGridSpec(
        num_scalar_prefetch=2,
        grid=(NT,),
        in_specs=[
            pl.BlockSpec((T, D), _gmap),
            pl.BlockSpec((1, 1, D), _emap3),
            pl.BlockSpec((1, 1, D), _emap3),
            pl.BlockSpec((1, D, H), _emap3),
            pl.BlockSpec((1, 1, H), _emap3),
            pl.BlockSpec((1, H, D), _emap3),
            pl.BlockSpec((1, 1, D), _emap3),
        ],
        out_specs=pl.BlockSpec((T, D), _gmap),
    )
    padded_out = pl.pallas_call(
        _ff_body,
        grid_spec=grid_spec,
        out_shape=jax.ShapeDtypeStruct((P, D), jnp.float32),
        compiler_params=pltpu.CompilerParams(
            dimension_semantics=("arbitrary",),
        ),
    )(
        tile_expert,
        total_tiles,
        padded_x,
        ln_g.reshape(E, 1, D),
        ln_b.reshape(E, 1, D),
        W1,
        b1.reshape(E, 1, H),
        W2,
        b2.reshape(E, 1, D),
    )

    out = _make_sc_gather(S, D, P, chunk, nw)(padded_out, dest)
    return out.reshape(x.shape)
